# baseline (speedup 1.0000x reference)
import functools

import jax
import jax.numpy as jnp
from jax import lax
from jax.experimental import pallas as pl
from jax.experimental.pallas import tpu as pltpu
from jax.experimental.pallas import tpu_sc as plsc

N = 10000
E = 320000
C = 128
NW = 32
NCH = E // C
NCHMAX = 81
RPS = 624


def _make_spmm(D):
    mesh = plsc.VectorSubcoreMesh(core_axis_name="c", subcore_axis_name="s")
    KV = D // 16

    @functools.partial(
        pl.kernel,
        out_type=jax.ShapeDtypeStruct((2 * N, D), jnp.float32),
        mesh=mesh,
        compiler_params=pltpu.CompilerParams(
            needs_layout_passes=False, use_tc_tiling_on_sc=False),
        scratch_types=[
            pltpu.VMEM((C,), jnp.int32),
            pltpu.VMEM((C,), jnp.int32),
            pltpu.VMEM((C,), jnp.int32),
            pltpu.VMEM((C,), jnp.int32),
            pltpu.VMEM((C,), jnp.int32),
            pltpu.VMEM((C,), jnp.int32),
            pltpu.VMEM((C,), jnp.int32),
            pltpu.VMEM((C,), jnp.int32),
            pltpu.VMEM((C,), jnp.int32),
            pltpu.VMEM((C,), jnp.float32),
            pltpu.VMEM((C,), jnp.float32),
            pltpu.VMEM((C,), jnp.float32),
            pltpu.VMEM((C, D), jnp.float32),
            pltpu.VMEM((C, D), jnp.float32),
            pltpu.VMEM((C, D), jnp.float32),
            pltpu.VMEM_SHARED((N, D), jnp.float32),
            pltpu.SemaphoreType.DMA,
            pltpu.SemaphoreType.DMA,
            pltpu.SemaphoreType.DMA,
            pltpu.SemaphoreType.DMA,
            pltpu.SemaphoreType.DMA,
            pltpu.SemaphoreType.DMA,
            pltpu.SemaphoreType.DMA,
            pltpu.SemaphoreType.DMA,
            pltpu.SemaphoreType.DMA,
        ],
    )
    def spmm(x_hbm, cols_hbm, rows_hbm, w_hbm, out_hbm,
             colv0, colv1, colv2, rowv0, rowv1, rowv2,
             ridx0, ridx1, ridx2, wv0, wv1, wv2, gbuf0, gbuf1, gbuf2, acc,
             isem0, isem1, isem2, gsem0, gsem1, gsem2, ssem0, ssem1, ssem2):
        c = lax.axis_index("c")
        s = lax.axis_index("s")
        wid = s * 2 + c
        r0 = s * RPS
        nch = (NCH - wid + NW - 1) // NW

        sets = ((colv0, rowv0, ridx0, wv0, gbuf0, isem0, gsem0, ssem0),
                (colv1, rowv1, ridx1, wv1, gbuf1, isem1, gsem1, ssem1),
                (colv2, rowv2, ridx2, wv2, gbuf2, isem2, gsem2, ssem2))

        def chunk_base(t):
            return (wid + NW * jnp.minimum(t, nch - 1)) * C

        def start_idx(t, st):
            colv, rowv, _, wv, _, isem, _, _ = st
            base = chunk_base(t)
            pltpu.async_copy(cols_hbm.at[pl.ds(base, C)], colv, isem)
            pltpu.async_copy(rows_hbm.at[pl.ds(base, C)], rowv, isem)
            pltpu.async_copy(w_hbm.at[pl.ds(base, C)], wv, isem)

        def wait_idx(t, st):
            colv, rowv, _, wv, _, isem, _, _ = st
            base = chunk_base(t)
            pltpu.make_async_copy(cols_hbm.at[pl.ds(base, C)], colv,
                                  isem).wait()
            pltpu.make_async_copy(rows_hbm.at[pl.ds(base, C)], rowv,
                                  isem).wait()
            pltpu.make_async_copy(w_hbm.at[pl.ds(base, C)], wv, isem).wait()

        def start_gather(st):
            colv, _, _, _, gbuf, _, gsem, _ = st
            pltpu.async_copy(x_hbm.at[colv], gbuf, gsem)

        def wait_gather(st):
            colv, _, _, _, gbuf, _, gsem, _ = st
            pltpu.make_async_copy(x_hbm.at[colv], gbuf, gsem).wait()

        def start_scatter(st):
            _, _, ridx, _, gbuf, _, _, ssem = st
            pltpu.async_copy(gbuf, acc.at[ridx], ssem, add=True)

        def wait_scatter(st):
            _, _, ridx, _, gbuf, _, _, ssem = st
            pltpu.make_async_copy(gbuf, acc.at[ridx], ssem).wait()

        def zrow(i, carry):
            for k in range(KV):
                gbuf2[i, pl.ds(k * 16, 16)] = jnp.zeros((16,), jnp.float32)
            return carry

        lax.fori_loop(0, C, zrow, 0, unroll=4)
        for q in range(RPS // C):
            pltpu.sync_copy(gbuf2, acc.at[pl.ds(r0 + q * C, C)])
        rem = RPS - (RPS // C) * C
        pltpu.sync_copy(gbuf2.at[pl.ds(0, rem)],
                        acc.at[pl.ds(r0 + (RPS // C) * C, rem)])

        @pl.when(s == 15)
        def _zero_tail():
            pltpu.sync_copy(gbuf2.at[pl.ds(0, N - 16 * RPS)],
                            acc.at[pl.ds(16 * RPS, N - 16 * RPS)])

        plsc.subcore_barrier()

        start_idx(0, sets[0])
        start_idx(1, sets[1])
        start_idx(2, sets[2])
        wait_idx(0, sets[0])
        start_gather(sets[0])

        def step(t, cur, nxt):
            colv, rowv, ridx, wv, gbuf, _, _, _ = cur
            wait_idx(t + 1, nxt)

            @pl.when(t >= 2)
            def _free_next_gbuf():
                wait_scatter(nxt)

            start_gather(nxt)
            wait_gather(cur)

            @pl.when(t >= nch)
            def _pad_zero():
                for k in range(8):
                    wv[pl.ds(k * 16, 16)] = jnp.zeros((16,), jnp.float32)

            def edge_body(e, carry):
                bw = plsc.load_gather(wv, [jnp.full((16,), e, jnp.int32)])
                for k in range(KV):
                    sl = pl.ds(k * 16, 16)
                    gbuf[e, sl] = gbuf[e, sl] * bw
                return carry

            lax.fori_loop(0, C, edge_body, 0, unroll=8)
            for k in range(8):
                sl = pl.ds(k * 16, 16)
                ridx[sl] = rowv[sl]
            start_scatter(cur)
            start_idx(t + 3, cur)

        def triple_body(u, carry):
            step(3 * u, sets[0], sets[1])
            step(3 * u + 1, sets[1], sets[2])
            step(3 * u + 2, sets[2], sets[0])
            return carry

        lax.fori_loop(0, NCHMAX // 3, triple_body, 0)

        wait_scatter(sets[(NCHMAX - 2) % 3])
        wait_scatter(sets[(NCHMAX - 1) % 3])
        wait_gather(sets[NCHMAX % 3])
        wait_idx(NCHMAX + 1, sets[(NCHMAX + 1) % 3])
        wait_idx(NCHMAX + 2, sets[(NCHMAX + 2) % 3])

        plsc.subcore_barrier()
        pltpu.sync_copy(acc.at[pl.ds(r0, RPS)],
                        out_hbm.at[pl.ds(c * N + r0, RPS)])

        @pl.when(s == 15)
        def _write_tail():
            pltpu.sync_copy(acc.at[pl.ds(16 * RPS, N - 16 * RPS)],
                            out_hbm.at[pl.ds(c * N + 16 * RPS, N - 16 * RPS)])

    return spmm


def _make_dense(Din, Dout, R):
    def body(p_ref, w_ref, b_ref, o_ref):
        x = p_ref[0] + p_ref[1]
        y = jnp.dot(x, w_ref[...], preferred_element_type=jnp.float32,
                    precision=lax.Precision.HIGHEST)
        y = y + b_ref[...]
        nrm = jnp.sqrt(jnp.sum(y * y, axis=1, keepdims=True))
        o_ref[...] = y / jnp.maximum(nrm, 1e-12)

    return pl.pallas_call(
        body,
        grid=(N // R,),
        in_specs=[
            pl.BlockSpec((2, R, Din), lambda i: (0, i, 0)),
            pl.BlockSpec((Din, Dout), lambda i: (0, 0)),
            pl.BlockSpec((1, Dout), lambda i: (0, 0)),
        ],
        out_specs=pl.BlockSpec((R, Dout), lambda i: (i, 0)),
        out_shape=jax.ShapeDtypeStruct((N, Dout), jnp.float32),
    )


_spmm_128 = _make_spmm(128)
_spmm_64 = _make_spmm(64)
_dense_0 = _make_dense(128, 64, 1000)
_dense_1 = _make_dense(64, 128, 1000)


def kernel(fts, edge_index, edge_weight, W_gc_0, b_gc_0, W_gc_1, b_gc_1):
    rows = edge_index[0]
    cols = edge_index[1]
    p0 = _spmm_128(fts, cols, rows, edge_weight).reshape(2, N, 128)
    ego = _dense_0(p0, W_gc_0, b_gc_0)
    p1 = _spmm_64(ego, cols, rows, edge_weight).reshape(2, N, 64)
    return _dense_1(p1, W_gc_1, b_gc_1)


# speedup vs baseline: 1.0073x; 1.0073x over previous
import functools

import jax
import jax.numpy as jnp
from jax import lax
from jax.experimental import pallas as pl
from jax.experimental.pallas import tpu as pltpu
from jax.experimental.pallas import tpu_sc as plsc

N = 10000
E = 320000
C = 128
NW = 32
NCH = E // C
NCHMAX = 81
RPS = 624


def _make_spmm(D):
    mesh = plsc.VectorSubcoreMesh(core_axis_name="c", subcore_axis_name="s")
    KV = D // 16

    @functools.partial(
        pl.kernel,
        out_type=jax.ShapeDtypeStruct((2 * N, D), jnp.float32),
        mesh=mesh,
        compiler_params=pltpu.CompilerParams(
            needs_layout_passes=False, use_tc_tiling_on_sc=False),
        scratch_types=[
            pltpu.VMEM((C,), jnp.int32),
            pltpu.VMEM((C,), jnp.int32),
            pltpu.VMEM((C,), jnp.int32),
            pltpu.VMEM((C,), jnp.int32),
            pltpu.VMEM((C,), jnp.int32),
            pltpu.VMEM((C,), jnp.int32),
            pltpu.VMEM((C,), jnp.int32),
            pltpu.VMEM((C,), jnp.int32),
            pltpu.VMEM((C,), jnp.int32),
            pltpu.VMEM((C,), jnp.float32),
            pltpu.VMEM((C,), jnp.float32),
            pltpu.VMEM((C,), jnp.float32),
            pltpu.VMEM((C, D), jnp.float32),
            pltpu.VMEM((C, D), jnp.float32),
            pltpu.VMEM((C, D), jnp.float32),
            pltpu.VMEM_SHARED((N, D), jnp.float32),
            pltpu.SemaphoreType.DMA,
            pltpu.SemaphoreType.DMA,
            pltpu.SemaphoreType.DMA,
            pltpu.SemaphoreType.DMA,
            pltpu.SemaphoreType.DMA,
            pltpu.SemaphoreType.DMA,
            pltpu.SemaphoreType.DMA,
            pltpu.SemaphoreType.DMA,
            pltpu.SemaphoreType.DMA,
        ],
    )
    def spmm(x_hbm, cols_hbm, rows_hbm, w_hbm, out_hbm,
             colv0, colv1, colv2, rowv0, rowv1, rowv2,
             ridx0, ridx1, ridx2, wv0, wv1, wv2, gbuf0, gbuf1, gbuf2, acc,
             isem0, isem1, isem2, gsem0, gsem1, gsem2, ssem0, ssem1, ssem2):
        c = lax.axis_index("c")
        s = lax.axis_index("s")
        wid = s * 2 + c
        r0 = s * RPS
        nch = (NCH - wid + NW - 1) // NW

        sets = ((colv0, rowv0, ridx0, wv0, gbuf0, isem0, gsem0, ssem0),
                (colv1, rowv1, ridx1, wv1, gbuf1, isem1, gsem1, ssem1),
                (colv2, rowv2, ridx2, wv2, gbuf2, isem2, gsem2, ssem2))

        def chunk_base(t):
            return (wid + NW * jnp.minimum(t, nch - 1)) * C

        def start_idx(t, st):
            colv, rowv, _, wv, _, isem, _, _ = st
            base = chunk_base(t)
            pltpu.async_copy(cols_hbm.at[pl.ds(base, C)], colv, isem)
            pltpu.async_copy(rows_hbm.at[pl.ds(base, C)], rowv, isem)
            pltpu.async_copy(w_hbm.at[pl.ds(base, C)], wv, isem)

        def wait_idx(t, st):
            colv, rowv, _, wv, _, isem, _, _ = st
            base = chunk_base(t)
            pltpu.make_async_copy(cols_hbm.at[pl.ds(base, C)], colv,
                                  isem).wait()
            pltpu.make_async_copy(rows_hbm.at[pl.ds(base, C)], rowv,
                                  isem).wait()
            pltpu.make_async_copy(w_hbm.at[pl.ds(base, C)], wv, isem).wait()

        def start_gather(st):
            colv, _, _, _, gbuf, _, gsem, _ = st
            pltpu.async_copy(x_hbm.at[colv], gbuf, gsem)

        def wait_gather(st):
            colv, _, _, _, gbuf, _, gsem, _ = st
            pltpu.make_async_copy(x_hbm.at[colv], gbuf, gsem).wait()

        def start_scatter(st):
            _, _, ridx, _, gbuf, _, _, ssem = st
            pltpu.async_copy(gbuf, acc.at[ridx], ssem, add=True)

        def wait_scatter(st):
            _, _, ridx, _, gbuf, _, _, ssem = st
            pltpu.make_async_copy(gbuf, acc.at[ridx], ssem).wait()

        def zrow(i, carry):
            for k in range(KV):
                gbuf2[i, pl.ds(k * 16, 16)] = jnp.zeros((16,), jnp.float32)
            return carry

        lax.fori_loop(0, C, zrow, 0, unroll=4)
        for q in range(RPS // C):
            pltpu.sync_copy(gbuf2, acc.at[pl.ds(r0 + q * C, C)])
        rem = RPS - (RPS // C) * C
        pltpu.sync_copy(gbuf2.at[pl.ds(0, rem)],
                        acc.at[pl.ds(r0 + (RPS // C) * C, rem)])

        @pl.when(s == 15)
        def _zero_tail():
            pltpu.sync_copy(gbuf2.at[pl.ds(0, N - 16 * RPS)],
                            acc.at[pl.ds(16 * RPS, N - 16 * RPS)])

        plsc.subcore_barrier()

        start_idx(0, sets[0])
        start_idx(1, sets[1])
        start_idx(2, sets[2])
        wait_idx(0, sets[0])
        start_gather(sets[0])

        def step(t, cur, nxt):
            colv, rowv, ridx, wv, gbuf, _, _, _ = cur
            wait_idx(t + 1, nxt)

            @pl.when(t >= 2)
            def _free_next_gbuf():
                wait_scatter(nxt)

            start_gather(nxt)
            wait_gather(cur)

            @pl.when(t >= nch)
            def _pad_zero():
                for k in range(8):
                    wv[pl.ds(k * 16, 16)] = jnp.zeros((16,), jnp.float32)

            def edge_body(e, carry):
                bw = plsc.load_gather(wv, [jnp.full((16,), e, jnp.int32)])
                for k in range(KV):
                    sl = pl.ds(k * 16, 16)
                    gbuf[e, sl] = gbuf[e, sl] * bw
                return carry

            lax.fori_loop(0, C, edge_body, 0, unroll=4)
            for k in range(8):
                sl = pl.ds(k * 16, 16)
                ridx[sl] = rowv[sl]
            start_scatter(cur)
            start_idx(t + 3, cur)

        def triple_body(u, carry):
            step(3 * u, sets[0], sets[1])
            step(3 * u + 1, sets[1], sets[2])
            step(3 * u + 2, sets[2], sets[0])
            return carry

        lax.fori_loop(0, NCHMAX // 3, triple_body, 0)

        wait_scatter(sets[(NCHMAX - 2) % 3])
        wait_scatter(sets[(NCHMAX - 1) % 3])
        wait_gather(sets[NCHMAX % 3])
        wait_idx(NCHMAX + 1, sets[(NCHMAX + 1) % 3])
        wait_idx(NCHMAX + 2, sets[(NCHMAX + 2) % 3])

        plsc.subcore_barrier()
        pltpu.sync_copy(acc.at[pl.ds(r0, RPS)],
                        out_hbm.at[pl.ds(c * N + r0, RPS)])

        @pl.when(s == 15)
        def _write_tail():
            pltpu.sync_copy(acc.at[pl.ds(16 * RPS, N - 16 * RPS)],
                            out_hbm.at[pl.ds(c * N + 16 * RPS, N - 16 * RPS)])

    return spmm


def _make_dense(Din, Dout, R):
    def body(p_ref, w_ref, b_ref, o_ref):
        x = p_ref[0] + p_ref[1]
        y = jnp.dot(x, w_ref[...], preferred_element_type=jnp.float32,
                    precision=lax.Precision.HIGHEST)
        y = y + b_ref[...]
        nrm = jnp.sqrt(jnp.sum(y * y, axis=1, keepdims=True))
        o_ref[...] = y / jnp.maximum(nrm, 1e-12)

    return pl.pallas_call(
        body,
        grid=(N // R,),
        in_specs=[
            pl.BlockSpec((2, R, Din), lambda i: (0, i, 0)),
            pl.BlockSpec((Din, Dout), lambda i: (0, 0)),
            pl.BlockSpec((1, Dout), lambda i: (0, 0)),
        ],
        out_specs=pl.BlockSpec((R, Dout), lambda i: (i, 0)),
        out_shape=jax.ShapeDtypeStruct((N, Dout), jnp.float32),
    )


_spmm_128 = _make_spmm(128)
_spmm_64 = _make_spmm(64)
_dense_0 = _make_dense(128, 64, 1000)
_dense_1 = _make_dense(64, 128, 1000)


def kernel(fts, edge_index, edge_weight, W_gc_0, b_gc_0, W_gc_1, b_gc_1):
    rows = edge_index[0]
    cols = edge_index[1]
    p0 = _spmm_128(fts, cols, rows, edge_weight).reshape(2, N, 128)
    ego = _dense_0(p0, W_gc_0, b_gc_0)
    p1 = _spmm_64(ego, cols, rows, edge_weight).reshape(2, N, 64)
    return _dense_1(p1, W_gc_1, b_gc_1)


# speedup vs baseline: 1.0221x; 1.0147x over previous
import functools

import jax
import jax.numpy as jnp
from jax import lax
from jax.experimental import pallas as pl
from jax.experimental.pallas import tpu as pltpu
from jax.experimental.pallas import tpu_sc as plsc

N = 10000
E = 320000
C = 128
NW = 32
NCH = E // C
RPS = 624


def _make_spmm(D, NBUF):
    mesh = plsc.VectorSubcoreMesh(core_axis_name="c", subcore_axis_name="s")
    KV = D // 16
    GP = NBUF - 2
    nchmax = ((79 + NBUF - 1) // NBUF) * NBUF

    per_set = [
        pltpu.VMEM((C,), jnp.int32),
        pltpu.VMEM((C,), jnp.int32),
        pltpu.VMEM((C,), jnp.int32),
        pltpu.VMEM((C,), jnp.float32),
        pltpu.VMEM((C, D), jnp.float32),
        pltpu.SemaphoreType.DMA,
        pltpu.SemaphoreType.DMA,
        pltpu.SemaphoreType.DMA,
    ]

    @functools.partial(
        pl.kernel,
        out_type=jax.ShapeDtypeStruct((2 * N, D), jnp.float32),
        mesh=mesh,
        compiler_params=pltpu.CompilerParams(
            needs_layout_passes=False, use_tc_tiling_on_sc=False),
        scratch_types=per_set * NBUF
        + [pltpu.VMEM_SHARED((N, D), jnp.float32)],
    )
    def spmm(x_hbm, cols_hbm, rows_hbm, w_hbm, out_hbm, *scratch):
        sets = tuple(tuple(scratch[8 * i:8 * i + 8]) for i in range(NBUF))
        acc = scratch[8 * NBUF]
        c = lax.axis_index("c")
        s = lax.axis_index("s")
        wid = s * 2 + c
        r0 = s * RPS
        nch = (NCH - wid + NW - 1) // NW

        def chunk_base(t):
            return (wid + NW * jnp.minimum(t, nch - 1)) * C

        def start_idx(t, st):
            colv, rowv, _, wv, _, isem, _, _ = st
            base = chunk_base(t)
            pltpu.async_copy(cols_hbm.at[pl.ds(base, C)], colv, isem)
            pltpu.async_copy(rows_hbm.at[pl.ds(base, C)], rowv, isem)
            pltpu.async_copy(w_hbm.at[pl.ds(base, C)], wv, isem)

        def wait_idx(t, st):
            colv, rowv, _, wv, _, isem, _, _ = st
            base = chunk_base(t)
            pltpu.make_async_copy(cols_hbm.at[pl.ds(base, C)], colv,
                                  isem).wait()
            pltpu.make_async_copy(rows_hbm.at[pl.ds(base, C)], rowv,
                                  isem).wait()
            pltpu.make_async_copy(w_hbm.at[pl.ds(base, C)], wv, isem).wait()

        def start_gather(st):
            colv, _, _, _, gbuf, _, gsem, _ = st
            pltpu.async_copy(x_hbm.at[colv], gbuf, gsem)

        def wait_gather(st):
            colv, _, _, _, gbuf, _, gsem, _ = st
            pltpu.make_async_copy(x_hbm.at[colv], gbuf, gsem).wait()

        def start_scatter(st):
            _, _, ridx, _, gbuf, _, _, ssem = st
            pltpu.async_copy(gbuf, acc.at[ridx], ssem, add=True)

        def wait_scatter(st):
            _, _, ridx, _, gbuf, _, _, ssem = st
            pltpu.make_async_copy(gbuf, acc.at[ridx], ssem).wait()

        zbuf = sets[NBUF - 1][4]

        def zrow(i, carry):
            for k in range(KV):
                zbuf[i, pl.ds(k * 16, 16)] = jnp.zeros((16,), jnp.float32)
            return carry

        lax.fori_loop(0, C, zrow, 0, unroll=4)
        for q in range(RPS // C):
            pltpu.sync_copy(zbuf, acc.at[pl.ds(r0 + q * C, C)])
        rem = RPS - (RPS // C) * C
        pltpu.sync_copy(zbuf.at[pl.ds(0, rem)],
                        acc.at[pl.ds(r0 + (RPS // C) * C, rem)])

        @pl.when(s == 15)
        def _zero_tail():
            pltpu.sync_copy(zbuf.at[pl.ds(0, N - 16 * RPS)],
                            acc.at[pl.ds(16 * RPS, N - 16 * RPS)])

        plsc.subcore_barrier()

        for i in range(NBUF):
            start_idx(i, sets[i])
        for i in range(GP):
            wait_idx(i, sets[i])
            start_gather(sets[i])

        def step(t, i):
            cur = sets[i]
            nxg = sets[(i + GP) % NBUF]
            colv, rowv, ridx, wv, gbuf, _, _, _ = cur
            wait_idx(t + GP, nxg)

            @pl.when(t >= 2)
            def _free_next_gbuf():
                wait_scatter(nxg)

            start_gather(nxg)
            wait_gather(cur)

            @pl.when(t >= nch)
            def _pad_zero():
                for k in range(8):
                    wv[pl.ds(k * 16, 16)] = jnp.zeros((16,), jnp.float32)

            def edge_body(e, carry):
                bw = plsc.load_gather(wv, [jnp.full((16,), e, jnp.int32)])
                for k in range(KV):
                    sl = pl.ds(k * 16, 16)
                    gbuf[e, sl] = gbuf[e, sl] * bw
                return carry

            lax.fori_loop(0, C, edge_body, 0, unroll=4)
            for k in range(8):
                sl = pl.ds(k * 16, 16)
                ridx[sl] = rowv[sl]
            start_scatter(cur)
            start_idx(t + NBUF, cur)

        def ring_body(u, carry):
            for i in range(NBUF):
                step(NBUF * u + i, i)
            return carry

        lax.fori_loop(0, nchmax // NBUF, ring_body, 0)

        wait_scatter(sets[(nchmax - 2) % NBUF])
        wait_scatter(sets[(nchmax - 1) % NBUF])
        for i in range(GP):
            wait_gather(sets[(nchmax + i) % NBUF])
        for t in range(nchmax + GP, nchmax + NBUF):
            wait_idx(t, sets[t % NBUF])

        plsc.subcore_barrier()
        pltpu.sync_copy(acc.at[pl.ds(r0, RPS)],
                        out_hbm.at[pl.ds(c * N + r0, RPS)])

        @pl.when(s == 15)
        def _write_tail():
            pltpu.sync_copy(acc.at[pl.ds(16 * RPS, N - 16 * RPS)],
                            out_hbm.at[pl.ds(c * N + 16 * RPS, N - 16 * RPS)])

    return spmm


def _make_dense(Din, Dout, R):
    def body(p_ref, w_ref, b_ref, o_ref):
        x = p_ref[0] + p_ref[1]
        y = jnp.dot(x, w_ref[...], preferred_element_type=jnp.float32,
                    precision=lax.Precision.HIGHEST)
        y = y + b_ref[...]
        nrm = jnp.sqrt(jnp.sum(y * y, axis=1, keepdims=True))
        o_ref[...] = y / jnp.maximum(nrm, 1e-12)

    return pl.pallas_call(
        body,
        grid=(N // R,),
        in_specs=[
            pl.BlockSpec((2, R, Din), lambda i: (0, i, 0)),
            pl.BlockSpec((Din, Dout), lambda i: (0, 0)),
            pl.BlockSpec((1, Dout), lambda i: (0, 0)),
        ],
        out_specs=pl.BlockSpec((R, Dout), lambda i: (i, 0)),
        out_shape=jax.ShapeDtypeStruct((N, Dout), jnp.float32),
    )


_spmm_128 = _make_spmm(128, 3)
_spmm_64 = _make_spmm(64, 4)
_dense_0 = _make_dense(128, 64, 1000)
_dense_1 = _make_dense(64, 128, 1000)


def kernel(fts, edge_index, edge_weight, W_gc_0, b_gc_0, W_gc_1, b_gc_1):
    rows = edge_index[0]
    cols = edge_index[1]
    p0 = _spmm_128(fts, cols, rows, edge_weight).reshape(2, N, 128)
    ego = _dense_0(p0, W_gc_0, b_gc_0)
    p1 = _spmm_64(ego, cols, rows, edge_weight).reshape(2, N, 64)
    return _dense_1(p1, W_gc_1, b_gc_1)


# speedup vs baseline: 1.0405x; 1.0180x over previous
import functools

import jax
import jax.numpy as jnp
from jax import lax
from jax.experimental import pallas as pl
from jax.experimental.pallas import tpu as pltpu
from jax.experimental.pallas import tpu_sc as plsc

N = 10000
E = 320000
C = 128
NW = 32
NCH = E // C
RPS = 624


def _make_spmm(D, NBUF):
    mesh = plsc.VectorSubcoreMesh(core_axis_name="c", subcore_axis_name="s")
    KV = D // 16
    GP = NBUF - 2
    nchmax = ((79 + NBUF - 1) // NBUF) * NBUF

    per_set = [
        pltpu.VMEM((C,), jnp.int32),
        pltpu.VMEM((C,), jnp.int32),
        pltpu.VMEM((C,), jnp.int32),
        pltpu.VMEM((C,), jnp.float32),
        pltpu.VMEM((C, D), jnp.float32),
        pltpu.SemaphoreType.DMA,
        pltpu.SemaphoreType.DMA,
        pltpu.SemaphoreType.DMA,
    ]

    @functools.partial(
        pl.kernel,
        out_type=jax.ShapeDtypeStruct((2 * N, D), jnp.float32),
        mesh=mesh,
        compiler_params=pltpu.CompilerParams(
            needs_layout_passes=False, use_tc_tiling_on_sc=False),
        scratch_types=per_set * NBUF
        + [pltpu.VMEM_SHARED((N, D), jnp.float32)],
    )
    def spmm(x_hbm, cols_hbm, rows_hbm, w_hbm, out_hbm, *scratch):
        sets = tuple(tuple(scratch[8 * i:8 * i + 8]) for i in range(NBUF))
        acc = scratch[8 * NBUF]
        c = lax.axis_index("c")
        s = lax.axis_index("s")
        wid = s * 2 + c
        r0 = s * RPS
        nch = (NCH - wid + NW - 1) // NW

        def chunk_base(t):
            return (wid + NW * jnp.minimum(t, nch - 1)) * C

        def start_idx(t, st):
            colv, rowv, _, wv, _, isem, _, _ = st
            base = chunk_base(t)
            pltpu.async_copy(cols_hbm.at[pl.ds(base, C)], colv, isem)
            pltpu.async_copy(rows_hbm.at[pl.ds(base, C)], rowv, isem)
            pltpu.async_copy(w_hbm.at[pl.ds(base, C)], wv, isem)

        def wait_idx(t, st):
            colv, rowv, _, wv, _, isem, _, _ = st
            base = chunk_base(t)
            pltpu.make_async_copy(cols_hbm.at[pl.ds(base, C)], colv,
                                  isem).wait()
            pltpu.make_async_copy(rows_hbm.at[pl.ds(base, C)], rowv,
                                  isem).wait()
            pltpu.make_async_copy(w_hbm.at[pl.ds(base, C)], wv, isem).wait()

        def start_gather(st):
            colv, _, _, _, gbuf, _, gsem, _ = st
            pltpu.async_copy(x_hbm.at[colv], gbuf, gsem)

        def wait_gather(st):
            colv, _, _, _, gbuf, _, gsem, _ = st
            pltpu.make_async_copy(x_hbm.at[colv], gbuf, gsem).wait()

        def start_scatter(st):
            _, _, ridx, _, gbuf, _, _, ssem = st
            pltpu.async_copy(gbuf, acc.at[ridx], ssem, add=True)

        def wait_scatter(st):
            _, _, ridx, _, gbuf, _, _, ssem = st
            pltpu.make_async_copy(gbuf, acc.at[ridx], ssem).wait()

        zbuf = sets[NBUF - 1][4]

        def zrow(i, carry):
            for k in range(KV):
                zbuf[i, pl.ds(k * 16, 16)] = jnp.zeros((16,), jnp.float32)
            return carry

        lax.fori_loop(0, C, zrow, 0, unroll=4)
        for q in range(RPS // C):
            pltpu.sync_copy(zbuf, acc.at[pl.ds(r0 + q * C, C)])
        rem = RPS - (RPS // C) * C
        pltpu.sync_copy(zbuf.at[pl.ds(0, rem)],
                        acc.at[pl.ds(r0 + (RPS // C) * C, rem)])

        @pl.when(s == 15)
        def _zero_tail():
            pltpu.sync_copy(zbuf.at[pl.ds(0, N - 16 * RPS)],
                            acc.at[pl.ds(16 * RPS, N - 16 * RPS)])

        plsc.subcore_barrier()

        for i in range(NBUF):
            start_idx(i, sets[i])
        for i in range(GP):
            wait_idx(i, sets[i])
            start_gather(sets[i])

        def step(t, i):
            cur = sets[i]
            nxg = sets[(i + GP) % NBUF]
            colv, rowv, ridx, wv, gbuf, _, _, _ = cur

            @pl.when((t >= 2) & (t - 2 < nch))
            def _free_next_gbuf():
                wait_scatter(nxg)

            @pl.when(t + GP < nch)
            def _prefetch_gather():
                wait_idx(t + GP, nxg)
                start_gather(nxg)

            @pl.when(t < nch)
            def _process():
                wait_gather(cur)

                def edge_body(e, carry):
                    bw = plsc.load_gather(wv, [jnp.full((16,), e, jnp.int32)])
                    for k in range(KV):
                        sl = pl.ds(k * 16, 16)
                        gbuf[e, sl] = gbuf[e, sl] * bw
                    return carry

                lax.fori_loop(0, C, edge_body, 0, unroll=4)
                for k in range(8):
                    sl = pl.ds(k * 16, 16)
                    ridx[sl] = rowv[sl]
                start_scatter(cur)

            @pl.when(t + NBUF < nch)
            def _prefetch_idx():
                start_idx(t + NBUF, cur)

        def ring_body(u, carry):
            for i in range(NBUF):
                step(NBUF * u + i, i)
            return carry

        lax.fori_loop(0, nchmax // NBUF, ring_body, 0)

        for t in range(nchmax - 2, 79):
            _st = sets[t % NBUF]

            @pl.when(t < nch)
            def _drain(_st=_st):
                wait_scatter(_st)

        plsc.subcore_barrier()
        pltpu.sync_copy(acc.at[pl.ds(r0, RPS)],
                        out_hbm.at[pl.ds(c * N + r0, RPS)])

        @pl.when(s == 15)
        def _write_tail():
            pltpu.sync_copy(acc.at[pl.ds(16 * RPS, N - 16 * RPS)],
                            out_hbm.at[pl.ds(c * N + 16 * RPS, N - 16 * RPS)])

    return spmm


def _make_dense(Din, Dout, R):
    def body(p_ref, w_ref, b_ref, o_ref):
        x = p_ref[0] + p_ref[1]
        y = jnp.dot(x, w_ref[...], preferred_element_type=jnp.float32,
                    precision=lax.Precision.HIGHEST)
        y = y + b_ref[...]
        nrm = jnp.sqrt(jnp.sum(y * y, axis=1, keepdims=True))
        o_ref[...] = y / jnp.maximum(nrm, 1e-12)

    return pl.pallas_call(
        body,
        grid=(N // R,),
        in_specs=[
            pl.BlockSpec((2, R, Din), lambda i: (0, i, 0)),
            pl.BlockSpec((Din, Dout), lambda i: (0, 0)),
            pl.BlockSpec((1, Dout), lambda i: (0, 0)),
        ],
        out_specs=pl.BlockSpec((R, Dout), lambda i: (i, 0)),
        out_shape=jax.ShapeDtypeStruct((N, Dout), jnp.float32),
    )


_spmm_128 = _make_spmm(128, 3)
_spmm_64 = _make_spmm(64, 4)
_dense_0 = _make_dense(128, 64, 1000)
_dense_1 = _make_dense(64, 128, 1000)


def kernel(fts, edge_index, edge_weight, W_gc_0, b_gc_0, W_gc_1, b_gc_1):
    rows = edge_index[0]
    cols = edge_index[1]
    p0 = _spmm_128(fts, cols, rows, edge_weight).reshape(2, N, 128)
    ego = _dense_0(p0, W_gc_0, b_gc_0)
    p1 = _spmm_64(ego, cols, rows, edge_weight).reshape(2, N, 64)
    return _dense_1(p1, W_gc_1, b_gc_1)


# speedup vs baseline: 1.0547x; 1.0137x over previous
import functools

import jax
import jax.numpy as jnp
from jax import lax
from jax.experimental import pallas as pl
from jax.experimental.pallas import tpu as pltpu
from jax.experimental.pallas import tpu_sc as plsc

N = 10000
E = 320000
C = 128
NW = 32
NCH = E // C
RPS = 624


def _make_spmm(D, NBUF):
    mesh = plsc.VectorSubcoreMesh(core_axis_name="c", subcore_axis_name="s")
    KV = D // 16
    GP = NBUF - 2
    nchmax = ((79 + NBUF - 1) // NBUF) * NBUF

    per_set = [
        pltpu.VMEM((3, C), jnp.int32),
        pltpu.VMEM((C,), jnp.int32),
        pltpu.VMEM((C, D), jnp.float32),
        pltpu.SemaphoreType.DMA,
        pltpu.SemaphoreType.DMA,
        pltpu.SemaphoreType.DMA,
    ]

    @functools.partial(
        pl.kernel,
        out_type=jax.ShapeDtypeStruct((2 * N, D), jnp.float32),
        mesh=mesh,
        compiler_params=pltpu.CompilerParams(
            needs_layout_passes=False, use_tc_tiling_on_sc=False),
        scratch_types=per_set * NBUF
        + [pltpu.VMEM_SHARED((N, D), jnp.float32)],
    )
    def spmm(x_hbm, idxw_hbm, out_hbm, *scratch):
        sets = tuple(tuple(scratch[6 * i:6 * i + 6]) for i in range(NBUF))
        acc = scratch[6 * NBUF]
        c = lax.axis_index("c")
        s = lax.axis_index("s")
        wid = s * 2 + c
        r0 = s * RPS
        nch = (NCH - wid + NW - 1) // NW

        def start_idx(t, st):
            ibuf, _, _, isem, _, _ = st
            pltpu.async_copy(idxw_hbm.at[wid + NW * t], ibuf, isem)

        def wait_idx(t, st):
            ibuf, _, _, isem, _, _ = st
            pltpu.make_async_copy(idxw_hbm.at[wid + NW * t], ibuf,
                                  isem).wait()

        def start_gather(st):
            ibuf, _, gbuf, _, gsem, _ = st
            pltpu.async_copy(x_hbm.at[ibuf.at[0]], gbuf, gsem)

        def wait_gather(st):
            ibuf, _, gbuf, _, gsem, _ = st
            pltpu.make_async_copy(x_hbm.at[ibuf.at[0]], gbuf, gsem).wait()

        def start_scatter(st):
            _, ridx, gbuf, _, _, ssem = st
            pltpu.async_copy(gbuf, acc.at[ridx], ssem, add=True)

        def wait_scatter(st):
            _, ridx, gbuf, _, _, ssem = st
            pltpu.make_async_copy(gbuf, acc.at[ridx], ssem).wait()

        zbuf = sets[NBUF - 1][2]

        def zrow(i, carry):
            for k in range(KV):
                zbuf[i, pl.ds(k * 16, 16)] = jnp.zeros((16,), jnp.float32)
            return carry

        lax.fori_loop(0, C, zrow, 0, unroll=4)
        for q in range(RPS // C):
            pltpu.sync_copy(zbuf, acc.at[pl.ds(r0 + q * C, C)])
        rem = RPS - (RPS // C) * C
        pltpu.sync_copy(zbuf.at[pl.ds(0, rem)],
                        acc.at[pl.ds(r0 + (RPS // C) * C, rem)])

        @pl.when(s == 15)
        def _zero_tail():
            pltpu.sync_copy(zbuf.at[pl.ds(0, N - 16 * RPS)],
                            acc.at[pl.ds(16 * RPS, N - 16 * RPS)])

        plsc.subcore_barrier()

        for i in range(NBUF):
            start_idx(i, sets[i])
        for i in range(GP):
            wait_idx(i, sets[i])
            start_gather(sets[i])

        def step(t, i):
            cur = sets[i]
            nxg = sets[(i + GP) % NBUF]
            ibuf, ridx, gbuf, _, _, _ = cur

            @pl.when((t >= 2) & (t - 2 < nch))
            def _free_next_gbuf():
                wait_scatter(nxg)

            @pl.when(t + GP < nch)
            def _prefetch_gather():
                wait_idx(t + GP, nxg)
                start_gather(nxg)

            @pl.when(t < nch)
            def _process():
                wait_gather(cur)

                two = jnp.full((16,), 2, jnp.int32)

                def edge_body(e, carry):
                    bw = plsc.bitcast(
                        plsc.load_gather(
                            ibuf, [two, jnp.full((16,), e, jnp.int32)]),
                        jnp.float32)
                    for k in range(KV):
                        sl = pl.ds(k * 16, 16)
                        gbuf[e, sl] = gbuf[e, sl] * bw
                    return carry

                lax.fori_loop(0, C, edge_body, 0, unroll=4)
                for k in range(8):
                    sl = pl.ds(k * 16, 16)
                    ridx[sl] = ibuf[1, sl]
                start_scatter(cur)

            @pl.when(t + NBUF < nch)
            def _prefetch_idx():
                start_idx(t + NBUF, cur)

        def ring_body(u, carry):
            for i in range(NBUF):
                step(NBUF * u + i, i)
            return carry

        lax.fori_loop(0, nchmax // NBUF, ring_body, 0)

        for t in range(nchmax - 2, 79):
            _st = sets[t % NBUF]

            @pl.when(t < nch)
            def _drain(_st=_st):
                wait_scatter(_st)

        plsc.subcore_barrier()
        pltpu.sync_copy(acc.at[pl.ds(r0, RPS)],
                        out_hbm.at[pl.ds(c * N + r0, RPS)])

        @pl.when(s == 15)
        def _write_tail():
            pltpu.sync_copy(acc.at[pl.ds(16 * RPS, N - 16 * RPS)],
                            out_hbm.at[pl.ds(c * N + 16 * RPS, N - 16 * RPS)])

    return spmm


def _make_dense(Din, Dout, R):
    def body(p_ref, w_ref, b_ref, o_ref):
        x = p_ref[0] + p_ref[1]
        y = jnp.dot(x, w_ref[...], preferred_element_type=jnp.float32,
                    precision=lax.Precision.HIGHEST)
        y = y + b_ref[...]
        nrm = jnp.sqrt(jnp.sum(y * y, axis=1, keepdims=True))
        o_ref[...] = y / jnp.maximum(nrm, 1e-12)

    return pl.pallas_call(
        body,
        grid=(N // R,),
        in_specs=[
            pl.BlockSpec((2, R, Din), lambda i: (0, i, 0)),
            pl.BlockSpec((Din, Dout), lambda i: (0, 0)),
            pl.BlockSpec((1, Dout), lambda i: (0, 0)),
        ],
        out_specs=pl.BlockSpec((R, Dout), lambda i: (i, 0)),
        out_shape=jax.ShapeDtypeStruct((N, Dout), jnp.float32),
    )


_spmm_128 = _make_spmm(128, 3)
_spmm_64 = _make_spmm(64, 4)
_dense_0 = _make_dense(128, 64, 1000)
_dense_1 = _make_dense(64, 128, 1000)


def kernel(fts, edge_index, edge_weight, W_gc_0, b_gc_0, W_gc_1, b_gc_1):
    idxw = jnp.stack(
        [edge_index[1].reshape(NCH, C),
         edge_index[0].reshape(NCH, C),
         lax.bitcast_convert_type(edge_weight, jnp.int32).reshape(NCH, C)],
        axis=1)
    p0 = _spmm_128(fts, idxw).reshape(2, N, 128)
    ego = _dense_0(p0, W_gc_0, b_gc_0)
    p1 = _spmm_64(ego, idxw).reshape(2, N, 64)
    return _dense_1(p1, W_gc_1, b_gc_1)
